# baseline (speedup 1.0000x reference)
import jax
import jax.numpy as jnp
from itertools import product as _prod
from jax.experimental import pallas as pl
from jax.experimental.pallas import tpu as pltpu

_VSZ = 0.1
_FD = 128
_NBLK = 2
_NTILE = 2000


def _structure(points):
    N = points.shape[0]
    coords = jnp.floor(points / _VSZ).astype(jnp.int64)
    coords = coords - coords.min(axis=0)
    mx = coords.max(axis=0) + 1
    mx1, mx2 = mx[1], mx[2]
    h = coords[:, 0] * (mx1 * mx2) + coords[:, 1] * mx2 + coords[:, 2]
    uh, inv = jnp.unique(h, return_inverse=True, size=N, fill_value=-1)
    inv = inv.reshape(-1).astype(jnp.int32)
    M = jnp.sum(uh >= 0).astype(jnp.int32)
    row_valid = jnp.arange(N, dtype=jnp.int32) < M
    c0 = uh // (mx1 * mx2)
    r = uh % (mx1 * mx2)
    c1 = r // mx2
    c2 = r % mx2
    vc = jnp.stack([c0, c1, c2], axis=1)
    big = jnp.iinfo(uh.dtype).max
    uh_s = jnp.where(row_valid, uh, big)
    neigh = []
    for off in _prod((-1, 0, 1), repeat=3):
        nc = vc + jnp.array(off, dtype=vc.dtype)
        valid = jnp.all((nc >= 0) & (nc < mx[None, :]), axis=1) & row_valid
        nh = nc[:, 0] * (mx1 * mx2) + nc[:, 1] * mx2 + nc[:, 2]
        pos = jnp.searchsorted(uh_s, nh).astype(jnp.int32)
        pos_c = jnp.clip(pos, 0, N - 1)
        hit = uh_s[pos_c] == nh
        neigh.append(jnp.where(valid & hit, pos_c, -1))
    return inv, M, jnp.stack(neigh, axis=0)


def _conv_mm(G, W):
    N = G.shape[1]
    nt = N // _NTILE

    def body(g_ref, w_ref, o_ref):
        k = pl.program_id(1)

        @pl.when(k == 0)
        def _():
            o_ref[...] = jnp.zeros_like(o_ref)

        o_ref[...] += jnp.dot(g_ref[0], w_ref[0],
                              preferred_element_type=jnp.float32)

    return pl.pallas_call(
        body,
        grid=(nt, 27),
        in_specs=[
            pl.BlockSpec((1, _NTILE, _FD), lambda i, k: (k, i, 0)),
            pl.BlockSpec((1, _FD, _FD), lambda i, k: (k, 0, 0)),
        ],
        out_specs=pl.BlockSpec((_NTILE, _FD), lambda i, k: (i, 0)),
        out_shape=jax.ShapeDtypeStruct((N, _FD), jnp.float32),
        compiler_params=pltpu.CompilerParams(
            dimension_semantics=("parallel", "arbitrary")),
    )(G, W)


def kernel(source_points, target_points, enc_W, enc_b, ln_g, ln_b, convW,
           bn_g, bn_b, h1_W, h1_b, h2_W, h2_b):
    sc = source_points - source_points.mean(axis=0, keepdims=True)
    tc = target_points - target_points.mean(axis=0, keepdims=True)
    s_inv, s_M, s_nb = _structure(sc)
    t_inv, t_M, t_nb = _structure(tc)
    Npad = source_points.shape[0]

    def encode(p):
        x = p @ enc_W + enc_b
        m = x.mean(axis=-1, keepdims=True)
        v = ((x - m) ** 2).mean(axis=-1, keepdims=True)
        x = (x - m) / jnp.sqrt(v + 1e-5) * ln_g + ln_b
        return jax.nn.relu(x)

    def vox_mean(feats, inv):
        s = jax.ops.segment_sum(feats, inv, num_segments=Npad)
        c = jax.ops.segment_sum(jnp.ones((feats.shape[0],), feats.dtype),
                                inv, num_segments=Npad)
        return s / jnp.where(c > 0, c, jnp.ones_like(c))[:, None]

    def subm(f, W, nb):
        G = jnp.where(nb[:, :, None] >= 0, f[jnp.clip(nb, 0)], 0.0)
        return _conv_mm(G, W)

    def bn(x, g, b, mask, Mf):
        m = jnp.where(mask[:, None], x, 0.0).sum(axis=0) / Mf
        v = jnp.where(mask[:, None], (x - m) ** 2, 0.0).sum(axis=0) / Mf
        return (x - m) / jnp.sqrt(v + 1e-5) * g + b

    def blocks(f, nb, mask, Mf):
        x = f
        for bi in range(_NBLK):
            idn = x
            y = jax.nn.relu(bn(subm(x, convW[bi, 0], nb),
                               bn_g[bi, 0], bn_b[bi, 0], mask, Mf))
            y = bn(subm(y, convW[bi, 1], nb),
                   bn_g[bi, 1], bn_b[bi, 1], mask, Mf) + idn
            x = jax.nn.relu(y)
        return x

    s_mask = jnp.arange(Npad, dtype=jnp.int32) < s_M
    t_mask = jnp.arange(Npad, dtype=jnp.int32) < t_M
    s_Mf = s_M.astype(jnp.float32)
    t_Mf = t_M.astype(jnp.float32)

    s_feats = blocks(vox_mean(encode(source_points), s_inv), s_nb, s_mask, s_Mf)
    t_feats = blocks(vox_mean(encode(target_points), t_inv), t_nb, t_mask, t_Mf)
    sg = jnp.where(s_mask[:, None], s_feats, -jnp.inf).max(axis=0)
    tg = jnp.where(t_mask[:, None], t_feats, -jnp.inf).max(axis=0)
    comb = sg + tg
    h = jax.nn.relu(comb @ h1_W + h1_b)
    return h @ h2_W + h2_b
